# baseline (speedup 1.0000x reference)
import jax
import jax.numpy as jnp
from jax import lax
from jax.experimental import pallas as pl
from jax.experimental.pallas import tpu as pltpu
from jax.experimental.pallas import tpu_sc as plsc

_B, _C, _H, _W = 8, 4, 512, 512
_NI = 16384
_L = 16
_NC, _NS = 2, 16
_NW = _NC * _NS
_K = 2048
_T = _NI // _K
_GL = 1024
_NG = _K // _GL

_TASKS = [(comp, chunk) for comp in (0, 1) for chunk in range(_T - 1, -1, -1)]


def _sc_body(pred_hbm, ints0_hbm, ints1_hbm, out_hbm,
             pxb0_v, pxd0_v, idxb0_v, idxd0_v, valsb0_v, valsd0_v,
             pxb1_v, pxd1_v, idxb1_v, idxd1_v, valsb1_v, valsd1_v,
             stage_v, gsems, psems):
    cid = lax.axis_index("c")
    sid = lax.axis_index("s")
    wid = sid * _NC + cid
    b = wid // _C
    cls = wid % _C

    lane = lax.iota(jnp.int32, _L)
    pbase = wid * (_H * _W)

    bufs = ((pxb0_v, pxd0_v, idxb0_v, idxd0_v, valsb0_v, valsd0_v),
            (pxb1_v, pxd1_v, idxb1_v, idxd1_v, valsb1_v, valsd1_v))
    ints = (ints0_hbm, ints1_hbm)

    c1 = (cls == 1).astype(jnp.int32)
    c2 = (cls == 2).astype(jnp.int32)
    c3 = (cls == 3).astype(jnp.int32)
    goods = (1 + c1 + 2 * c2, c1 + 2 * c2 + c3)

    def plane_copies(task_i):
        comp, chunk = _TASKS[task_i]
        p = task_i % 2
        pxb_v, pxd_v = bufs[p][0], bufs[p][1]
        n0 = chunk * _K
        return [
            pltpu.make_async_copy(
                ints[comp].at[b, cls, 0, :, pl.ds(n0, _K)], pxb_v,
                psems.at[p]),
            pltpu.make_async_copy(
                ints[comp].at[b, cls, 1, :, pl.ds(n0, _K)], pxd_v,
                psems.at[p]),
        ]

    def idx_compute(task_i):
        p = task_i % 2
        pxb_v, pxd_v, idxb_v, idxd_v = bufs[p][:4]

        def idx_body(j, _):
            s = pl.ds(j * _L, _L)
            idxb_v[s] = pxb_v[0, s] * _W + pxb_v[1, s] + pbase
            idxd_v[s] = pxd_v[0, s] * _W + pxd_v[1, s] + pbase
            return 0

        lax.fori_loop(0, _K // _L, idx_body, 0, unroll=8)

    def gather_copies(task_i):
        p = task_i % 2
        idxb_v, idxd_v, valsb_v, valsd_v = bufs[p][2:]
        return [
            pltpu.make_async_copy(
                pred_hbm.at[src.at[pl.ds(g * _GL, _GL)]],
                dst.at[pl.ds(g * _GL, _GL)], gsems.at[p])
            for g in range(_NG)
            for src, dst in ((idxb_v, valsb_v), (idxd_v, valsd_v))
        ]

    def consume(task_i, acc):
        comp, chunk = _TASKS[task_i]
        p = task_i % 2
        valsb_v, valsd_v = bufs[p][4], bufs[p][5]

        def sum_body(m, a):
            s = pl.ds(m * _L, _L)
            d = valsb_v[s] - valsd_v[s]
            return a + d * d

        acc = lax.fori_loop(0, _K // _L, sum_body, acc, unroll=8)
        if chunk == 0:
            d0 = valsb_v[pl.ds(0, _L)] - valsd_v[pl.ds(0, _L)]
            sq0 = d0 * d0
            acc = acc + jnp.where(lane < goods[comp], 1.0 - 2.0 * sq0, 0.0)
        return acc

    acc = jnp.zeros((_L,), jnp.float32)
    n_tasks = len(_TASKS)

    for cpy in plane_copies(0):
        cpy.start()
    for cpy in plane_copies(0):
        cpy.wait()
    idx_compute(0)
    for cpy in plane_copies(1):
        cpy.start()
    for cpy in gather_copies(0):
        cpy.start()

    for i in range(1, n_tasks):
        for cpy in plane_copies(i):
            cpy.wait()
        idx_compute(i)
        for cpy in gather_copies(i):
            cpy.start()
        for cpy in gather_copies(i - 1):
            cpy.wait()
        if i + 1 < n_tasks:
            for cpy in plane_copies(i + 1):
                cpy.start()
        acc = consume(i - 1, acc)

    for cpy in gather_copies(n_tasks - 1):
        cpy.wait()
    acc = consume(n_tasks - 1, acc)

    stage_v[...] = acc
    pltpu.sync_copy(stage_v, out_hbm.at[wid])


@jax.jit
def kernel(prediction, intervals_comp_0, intervals_comp_1):
    pred_flat = prediction.reshape(-1)
    it0 = jnp.transpose(intervals_comp_0, (0, 1, 3, 4, 2))
    it1 = jnp.transpose(intervals_comp_1, (0, 1, 3, 4, 2))
    mesh = plsc.VectorSubcoreMesh(core_axis_name="c", subcore_axis_name="s",
                                  num_cores=_NC, num_subcores=_NS)
    out = pl.kernel(
        _sc_body,
        out_type=jax.ShapeDtypeStruct((_NW, _L), jnp.float32),
        mesh=mesh,
        scratch_types=[
            pltpu.VMEM((2, _K), jnp.int32),
            pltpu.VMEM((2, _K), jnp.int32),
            pltpu.VMEM((_K,), jnp.int32),
            pltpu.VMEM((_K,), jnp.int32),
            pltpu.VMEM((_K,), jnp.float32),
            pltpu.VMEM((_K,), jnp.float32),
            pltpu.VMEM((2, _K), jnp.int32),
            pltpu.VMEM((2, _K), jnp.int32),
            pltpu.VMEM((_K,), jnp.int32),
            pltpu.VMEM((_K,), jnp.int32),
            pltpu.VMEM((_K,), jnp.float32),
            pltpu.VMEM((_K,), jnp.float32),
            pltpu.VMEM((_L,), jnp.float32),
            pltpu.SemaphoreType.DMA((2,)),
            pltpu.SemaphoreType.DMA((2,)),
        ],
    )(pred_flat, it0, it1)
    return jnp.sum(out)


# speedup vs baseline: 1.0588x; 1.0588x over previous
import jax
import jax.numpy as jnp
from jax import lax
from jax.experimental import pallas as pl
from jax.experimental.pallas import tpu as pltpu
from jax.experimental.pallas import tpu_sc as plsc

_B, _C, _H, _W = 8, 4, 512, 512
_NI = 16384
_L = 16
_NC, _NS = 2, 16
_NW = _NC * _NS
_K = 4096
_T = _NI // _K
_GL = 1024
_NG = _K // _GL

_TASKS = [(comp, chunk) for comp in (0, 1) for chunk in range(_T - 1, -1, -1)]


def _sc_body(pred_hbm, ints0_hbm, ints1_hbm, out_hbm,
             pxb0_v, pxd0_v, idxb0_v, idxd0_v, valsb0_v, valsd0_v,
             pxb1_v, pxd1_v, idxb1_v, idxd1_v, valsb1_v, valsd1_v,
             stage_v, gsems, psems):
    cid = lax.axis_index("c")
    sid = lax.axis_index("s")
    wid = sid * _NC + cid
    b = wid // _C
    cls = wid % _C

    lane = lax.iota(jnp.int32, _L)
    pbase = wid * (_H * _W)

    bufs = ((pxb0_v, pxd0_v, idxb0_v, idxd0_v, valsb0_v, valsd0_v),
            (pxb1_v, pxd1_v, idxb1_v, idxd1_v, valsb1_v, valsd1_v))
    ints = (ints0_hbm, ints1_hbm)

    c1 = (cls == 1).astype(jnp.int32)
    c2 = (cls == 2).astype(jnp.int32)
    c3 = (cls == 3).astype(jnp.int32)
    goods = (1 + c1 + 2 * c2, c1 + 2 * c2 + c3)

    def plane_copies(task_i):
        comp, chunk = _TASKS[task_i]
        p = task_i % 2
        pxb_v, pxd_v = bufs[p][0], bufs[p][1]
        n0 = chunk * _K
        return [
            pltpu.make_async_copy(
                ints[comp].at[b, cls, 0, :, pl.ds(n0, _K)], pxb_v,
                psems.at[p]),
            pltpu.make_async_copy(
                ints[comp].at[b, cls, 1, :, pl.ds(n0, _K)], pxd_v,
                psems.at[p]),
        ]

    def idx_compute(task_i):
        p = task_i % 2
        pxb_v, pxd_v, idxb_v, idxd_v = bufs[p][:4]

        def idx_body(j, _):
            s = pl.ds(j * _L, _L)
            idxb_v[s] = pxb_v[0, s] * _W + pxb_v[1, s] + pbase
            idxd_v[s] = pxd_v[0, s] * _W + pxd_v[1, s] + pbase
            return 0

        lax.fori_loop(0, _K // _L, idx_body, 0, unroll=4)

    def gather_copies(task_i):
        p = task_i % 2
        idxb_v, idxd_v, valsb_v, valsd_v = bufs[p][2:]
        return [
            pltpu.make_async_copy(
                pred_hbm.at[src.at[pl.ds(g * _GL, _GL)]],
                dst.at[pl.ds(g * _GL, _GL)], gsems.at[p])
            for g in range(_NG)
            for src, dst in ((idxb_v, valsb_v), (idxd_v, valsd_v))
        ]

    def consume(task_i, acc):
        comp, chunk = _TASKS[task_i]
        p = task_i % 2
        valsb_v, valsd_v = bufs[p][4], bufs[p][5]

        def sum_body(m, a):
            s = pl.ds(m * _L, _L)
            d = valsb_v[s] - valsd_v[s]
            return a + d * d

        acc = lax.fori_loop(0, _K // _L, sum_body, acc, unroll=4)
        if chunk == 0:
            d0 = valsb_v[pl.ds(0, _L)] - valsd_v[pl.ds(0, _L)]
            sq0 = d0 * d0
            acc = acc + jnp.where(lane < goods[comp], 1.0 - 2.0 * sq0, 0.0)
        return acc

    acc = jnp.zeros((_L,), jnp.float32)
    n_tasks = len(_TASKS)

    for cpy in plane_copies(0):
        cpy.start()
    for cpy in plane_copies(0):
        cpy.wait()
    idx_compute(0)
    for cpy in plane_copies(1):
        cpy.start()
    for cpy in gather_copies(0):
        cpy.start()

    for i in range(1, n_tasks):
        for cpy in plane_copies(i):
            cpy.wait()
        idx_compute(i)
        for cpy in gather_copies(i):
            cpy.start()
        for cpy in gather_copies(i - 1):
            cpy.wait()
        if i + 1 < n_tasks:
            for cpy in plane_copies(i + 1):
                cpy.start()
        acc = consume(i - 1, acc)

    for cpy in gather_copies(n_tasks - 1):
        cpy.wait()
    acc = consume(n_tasks - 1, acc)

    stage_v[...] = acc
    pltpu.sync_copy(stage_v, out_hbm.at[wid])


@jax.jit
def kernel(prediction, intervals_comp_0, intervals_comp_1):
    pred_flat = prediction.reshape(-1)
    it0 = jnp.transpose(intervals_comp_0, (0, 1, 3, 4, 2))
    it1 = jnp.transpose(intervals_comp_1, (0, 1, 3, 4, 2))
    mesh = plsc.VectorSubcoreMesh(core_axis_name="c", subcore_axis_name="s",
                                  num_cores=_NC, num_subcores=_NS)
    out = pl.kernel(
        _sc_body,
        out_type=jax.ShapeDtypeStruct((_NW, _L), jnp.float32),
        mesh=mesh,
        scratch_types=[
            pltpu.VMEM((2, _K), jnp.int32),
            pltpu.VMEM((2, _K), jnp.int32),
            pltpu.VMEM((_K,), jnp.int32),
            pltpu.VMEM((_K,), jnp.int32),
            pltpu.VMEM((_K,), jnp.float32),
            pltpu.VMEM((_K,), jnp.float32),
            pltpu.VMEM((2, _K), jnp.int32),
            pltpu.VMEM((2, _K), jnp.int32),
            pltpu.VMEM((_K,), jnp.int32),
            pltpu.VMEM((_K,), jnp.int32),
            pltpu.VMEM((_K,), jnp.float32),
            pltpu.VMEM((_K,), jnp.float32),
            pltpu.VMEM((_L,), jnp.float32),
            pltpu.SemaphoreType.DMA((2,)),
            pltpu.SemaphoreType.DMA((2,)),
        ],
    )(pred_flat, it0, it1)
    return jnp.sum(out)


# speedup vs baseline: 1.0628x; 1.0037x over previous
import jax
import jax.numpy as jnp
from jax import lax
from jax.experimental import pallas as pl
from jax.experimental.pallas import tpu as pltpu
from jax.experimental.pallas import tpu_sc as plsc

_B, _C, _H, _W = 8, 4, 512, 512
_NI = 16384
_L = 16
_NC, _NS = 2, 16
_NW = _NC * _NS
_K = 4096
_T = _NI // _K
_GL = 2048
_NG = _K // _GL

_TASKS = [(comp, chunk) for comp in (0, 1) for chunk in range(_T - 1, -1, -1)]


def _sc_body(pred_hbm, ints0_hbm, ints1_hbm, out_hbm,
             pxb0_v, pxd0_v, idxb0_v, idxd0_v, valsb0_v, valsd0_v,
             pxb1_v, pxd1_v, idxb1_v, idxd1_v, valsb1_v, valsd1_v,
             stage_v, gsems, psems):
    cid = lax.axis_index("c")
    sid = lax.axis_index("s")
    wid = sid * _NC + cid
    b = wid // _C
    cls = wid % _C

    lane = lax.iota(jnp.int32, _L)
    pbase = wid * (_H * _W)

    bufs = ((pxb0_v, pxd0_v, idxb0_v, idxd0_v, valsb0_v, valsd0_v),
            (pxb1_v, pxd1_v, idxb1_v, idxd1_v, valsb1_v, valsd1_v))
    ints = (ints0_hbm, ints1_hbm)

    c1 = (cls == 1).astype(jnp.int32)
    c2 = (cls == 2).astype(jnp.int32)
    c3 = (cls == 3).astype(jnp.int32)
    goods = (1 + c1 + 2 * c2, c1 + 2 * c2 + c3)

    def plane_copies(task_i):
        comp, chunk = _TASKS[task_i]
        p = task_i % 2
        pxb_v, pxd_v = bufs[p][0], bufs[p][1]
        n0 = chunk * _K
        return [
            pltpu.make_async_copy(
                ints[comp].at[b, cls, 0, :, pl.ds(n0, _K)], pxb_v,
                psems.at[p]),
            pltpu.make_async_copy(
                ints[comp].at[b, cls, 1, :, pl.ds(n0, _K)], pxd_v,
                psems.at[p]),
        ]

    def idx_compute(task_i):
        p = task_i % 2
        pxb_v, pxd_v, idxb_v, idxd_v = bufs[p][:4]

        def idx_body(j, _):
            s = pl.ds(j * _L, _L)
            idxb_v[s] = pxb_v[0, s] * _W + pxb_v[1, s] + pbase
            idxd_v[s] = pxd_v[0, s] * _W + pxd_v[1, s] + pbase
            return 0

        lax.fori_loop(0, _K // _L, idx_body, 0, unroll=4)

    def gather_copies(task_i):
        p = task_i % 2
        idxb_v, idxd_v, valsb_v, valsd_v = bufs[p][2:]
        return [
            pltpu.make_async_copy(
                pred_hbm.at[src.at[pl.ds(g * _GL, _GL)]],
                dst.at[pl.ds(g * _GL, _GL)], gsems.at[p])
            for g in range(_NG)
            for src, dst in ((idxb_v, valsb_v), (idxd_v, valsd_v))
        ]

    def consume(task_i, acc):
        comp, chunk = _TASKS[task_i]
        p = task_i % 2
        valsb_v, valsd_v = bufs[p][4], bufs[p][5]

        def sum_body(m, a):
            s = pl.ds(m * _L, _L)
            d = valsb_v[s] - valsd_v[s]
            return a + d * d

        acc = lax.fori_loop(0, _K // _L, sum_body, acc, unroll=4)
        if chunk == 0:
            d0 = valsb_v[pl.ds(0, _L)] - valsd_v[pl.ds(0, _L)]
            sq0 = d0 * d0
            acc = acc + jnp.where(lane < goods[comp], 1.0 - 2.0 * sq0, 0.0)
        return acc

    acc = jnp.zeros((_L,), jnp.float32)
    n_tasks = len(_TASKS)

    for cpy in plane_copies(0):
        cpy.start()
    for cpy in plane_copies(0):
        cpy.wait()
    idx_compute(0)
    for cpy in plane_copies(1):
        cpy.start()
    for cpy in gather_copies(0):
        cpy.start()

    for i in range(1, n_tasks):
        for cpy in plane_copies(i):
            cpy.wait()
        idx_compute(i)
        for cpy in gather_copies(i):
            cpy.start()
        for cpy in gather_copies(i - 1):
            cpy.wait()
        if i + 1 < n_tasks:
            for cpy in plane_copies(i + 1):
                cpy.start()
        acc = consume(i - 1, acc)

    for cpy in gather_copies(n_tasks - 1):
        cpy.wait()
    acc = consume(n_tasks - 1, acc)

    stage_v[...] = acc
    pltpu.sync_copy(stage_v, out_hbm.at[wid])


@jax.jit
def kernel(prediction, intervals_comp_0, intervals_comp_1):
    pred_flat = prediction.reshape(-1)
    it0 = jnp.transpose(intervals_comp_0, (0, 1, 3, 4, 2))
    it1 = jnp.transpose(intervals_comp_1, (0, 1, 3, 4, 2))
    mesh = plsc.VectorSubcoreMesh(core_axis_name="c", subcore_axis_name="s",
                                  num_cores=_NC, num_subcores=_NS)
    out = pl.kernel(
        _sc_body,
        out_type=jax.ShapeDtypeStruct((_NW, _L), jnp.float32),
        mesh=mesh,
        scratch_types=[
            pltpu.VMEM((2, _K), jnp.int32),
            pltpu.VMEM((2, _K), jnp.int32),
            pltpu.VMEM((_K,), jnp.int32),
            pltpu.VMEM((_K,), jnp.int32),
            pltpu.VMEM((_K,), jnp.float32),
            pltpu.VMEM((_K,), jnp.float32),
            pltpu.VMEM((2, _K), jnp.int32),
            pltpu.VMEM((2, _K), jnp.int32),
            pltpu.VMEM((_K,), jnp.int32),
            pltpu.VMEM((_K,), jnp.int32),
            pltpu.VMEM((_K,), jnp.float32),
            pltpu.VMEM((_K,), jnp.float32),
            pltpu.VMEM((_L,), jnp.float32),
            pltpu.SemaphoreType.DMA((2,)),
            pltpu.SemaphoreType.DMA((2,)),
        ],
    )(pred_flat, it0, it1)
    return jnp.sum(out)
